# Initial kernel scaffold; baseline (speedup 1.0000x reference)
import jax
import jax.numpy as jnp
from jax.experimental import pallas as pl


def kernel(a_feat, a_seg_ids, b_seg_ids, num_segments):
    raise NotImplementedError("write your pallas kernel here")


# speedup vs baseline: 2.9468x; 2.9468x over previous
import functools

import jax
import jax.numpy as jnp
from jax import lax
from jax.experimental import pallas as pl
from jax.experimental.pallas import tpu as pltpu
from jax.experimental.pallas import tpu_sc as plsc

N_A = 320000
N_B = 320000
D = 128
NUM_SEG = 40000
NJ = D // 16

NC = 2
NS = 16
NW = NC * NS

SEG_CHUNK = 200
NUM_CHUNKS = NUM_SEG // SEG_CHUNK
CHUNK_ITERS = -(-NUM_CHUNKS // NW)
RB = 256
RB_PAD = RB + 8
IDS_PAD = RB + 24
STARTS_PAD = 208

BGC = 80
BPW = N_B // NW
GPW = BPW // BGC
NBUF = 5
NGROUP = GPW // NBUF

_mesh = plsc.VectorSubcoreMesh(core_axis_name="c", subcore_axis_name="s")


@functools.partial(
    pl.kernel,
    out_type=jax.ShapeDtypeStruct((NUM_SEG * D,), jnp.float32),
    mesh=_mesh,
    scratch_types=[
        pltpu.VMEM((RB_PAD * D,), jnp.float32),
        pltpu.VMEM((IDS_PAD,), jnp.int32),
        pltpu.VMEM((STARTS_PAD,), jnp.int32),
        pltpu.VMEM((SEG_CHUNK * D,), jnp.float32),
    ],
)
def _segmax(a_hbm, ids_hbm, starts_hbm, pooled_hbm,
            row_buf, ids_vmem, starts_vmem, out_buf):
    wid = lax.axis_index("s") * NC + lax.axis_index("c")
    pltpu.sync_copy(starts_hbm, starts_vmem)

    zero16 = jnp.zeros((16,), jnp.float32)

    for t in range(CHUNK_ITERS):
        c = wid + t * NW

        @pl.when(c < NUM_CHUNKS)
        def _(c=c):
            sv = starts_vmem[pl.ds(c, 16)]
            lo = sv[0]
            hi = sv[1]
            c0 = c * SEG_CHUNK

            def zero_body(s, carry):
                out_buf[pl.ds(s * 16, 16)] = zero16
                return carry

            lax.fori_loop(0, SEG_CHUNK * NJ, zero_body, 0)

            nrows = hi - lo
            nblocks = (nrows + RB - 1) // RB

            def block_body(b, carry, lo=lo, hi=hi, c0=c0):
                base = lo + b * RB
                n = jnp.minimum(RB, hi - base)
                win = jnp.minimum((base // 8) * 8, N_A - RB_PAD)
                skew = base - win
                pltpu.sync_copy(ids_hbm.at[pl.ds(win, IDS_PAD)], ids_vmem)
                pltpu.sync_copy(a_hbm.at[pl.ds(win * D, RB_PAD * D)],
                                row_buf)

                def row_body(i, rc, skew=skew, c0=c0):
                    pid = rc[0]
                    acc = rc[1:]
                    sid = ids_vmem[pl.ds(skew + i, 16)][0]
                    is_new = sid != pid

                    @pl.when(is_new & (pid >= 0))
                    def _():
                        for j in range(NJ):
                            out_buf[pl.ds((pid - c0) * D + j * 16, 16)] = acc[j]

                    new_acc = []
                    for j in range(NJ):
                        v = row_buf[pl.ds((skew + i) * D + j * 16, 16)]
                        new_acc.append(
                            jnp.where(is_new, v, jnp.maximum(acc[j], v)))
                    return (sid,) + tuple(new_acc)

                return lax.fori_loop(0, n, row_body, carry)

            init = (jnp.int32(-1),) + tuple(
                jnp.full((16,), -jnp.inf, jnp.float32) for _ in range(NJ))
            fin = lax.fori_loop(0, nblocks, block_body, init)
            last_id = fin[0]
            last_acc = fin[1:]

            @pl.when(last_id >= 0)
            def _(last_id=last_id, last_acc=last_acc, c0=c0):
                for j in range(NJ):
                    out_buf[pl.ds((last_id - c0) * D + j * 16, 16)] = last_acc[j]

            pltpu.sync_copy(out_buf,
                            pooled_hbm.at[pl.ds(c0 * D, SEG_CHUNK * D)])


@functools.partial(
    pl.kernel,
    out_type=jax.ShapeDtypeStruct((N_B, D), jnp.float32),
    mesh=_mesh,
    scratch_types=[
        pltpu.VMEM((BPW,), jnp.int32),
        pltpu.VMEM((NBUF, BGC, D), jnp.float32),
        pltpu.SemaphoreType.DMA,
        pltpu.SemaphoreType.DMA,
    ],
)
def _unpool(pooled_hbm, bids_hbm, out_hbm, idx_all, rows, gsem, wsem):
    wid = lax.axis_index("s") * NC + lax.axis_index("c")
    pltpu.sync_copy(bids_hbm.at[pl.ds(wid * BPW, BPW)], idx_all)

    def group_body(g, carry):
        ghandles = []
        for b in range(NBUF):
            k = g * NBUF + b
            ghandles.append(
                pltpu.async_copy(pooled_hbm.at[idx_all.at[pl.ds(k * BGC, BGC)]],
                                 rows.at[b], gsem))
        whandles = []
        for b in range(NBUF):
            k = g * NBUF + b
            ghandles[b].wait()
            off = wid * BPW + k * BGC
            whandles.append(
                pltpu.async_copy(rows.at[b], out_hbm.at[pl.ds(off, BGC)], wsem))
        for wh in whandles:
            wh.wait()
        return carry

    lax.fori_loop(0, NGROUP, group_body, 0)


def kernel(a_feat, a_seg_ids, b_seg_ids, num_segments):
    del num_segments
    bounds = jnp.arange(0, NUM_SEG + 1, SEG_CHUNK, dtype=jnp.int32)
    starts = jnp.searchsorted(a_seg_ids, bounds, side="left").astype(jnp.int32)
    starts = jnp.concatenate(
        [starts, jnp.full((STARTS_PAD - NUM_CHUNKS - 1,), N_A, jnp.int32)])
    pooled = _segmax(a_feat.reshape(N_A * D), a_seg_ids, starts)
    return _unpool(pooled.reshape(NUM_SEG, D), b_seg_ids)
